# baseline (speedup 1.0000x reference)
import functools

import jax
import jax.numpy as jnp
from jax import lax
from jax.experimental import pallas as pl
from jax.experimental.pallas import tpu as pltpu
from jax.experimental.pallas import tpu_sc as plsc

NC = 2
NS = 16
D = 16
CHUNK = 2000


def _segment_sum_sc(bonds, src, nbr, init, n_nodes):
    n_edges = src.shape[0]
    n_workers = NC * NS
    epw = n_edges // n_workers
    n_chunks = epw // CHUNK
    n_pad = ((n_nodes + NS * 8 - 1) // (NS * 8)) * (NS * 8)
    rows_per_tile = n_pad // NS

    mesh = plsc.VectorSubcoreMesh(core_axis_name="c", subcore_axis_name="s",
                                  num_cores=NC, num_subcores=NS)

    @functools.partial(
        pl.kernel,
        out_type=jax.ShapeDtypeStruct((NC, n_pad, D), jnp.float32),
        mesh=mesh,
        compiler_params=pltpu.CompilerParams(use_tc_tiling_on_sc=False,
                                             needs_layout_passes=False),
        scratch_types=[
            pltpu.VMEM((CHUNK,), jnp.int32),
            pltpu.VMEM((CHUNK,), jnp.int32),
            pltpu.VMEM((CHUNK,), jnp.int32),
            pltpu.VMEM((CHUNK,), jnp.int32),
            pltpu.VMEM((CHUNK, D), jnp.float32),
            pltpu.VMEM((CHUNK, D), jnp.float32),
            pltpu.VMEM_SHARED((n_pad, D), jnp.float32),
            pltpu.SemaphoreType.DMA,
            pltpu.SemaphoreType.DMA,
        ],
    )
    def seg_kernel(bonds_hbm, src_hbm, nbr_hbm, init_hbm, out_hbm,
                   nbr_v0, nbr_v1, src_v0, src_v1, rows_v0, rows_v1,
                   acc_sh, sem0, sem1):
        nbr_vs = (nbr_v0, nbr_v1)
        src_vs = (src_v0, src_v1)
        rows_vs = (rows_v0, rows_v1)
        sems = (sem0, sem1)
        c = lax.axis_index("c")
        s = lax.axis_index("s")
        r0 = s * rows_per_tile
        pltpu.sync_copy(init_hbm.at[c, pl.ds(r0, rows_per_tile)],
                        acc_sh.at[pl.ds(r0, rows_per_tile)])
        plsc.subcore_barrier()
        base0 = (c * NS + s) * epw

        def load_and_gather(k):
            b = k % 2
            base = base0 + k * CHUNK
            pltpu.sync_copy(nbr_hbm.at[pl.ds(base, CHUNK)], nbr_vs[b])
            pltpu.sync_copy(src_hbm.at[pl.ds(base, CHUNK)], src_vs[b])
            return pltpu.async_copy(bonds_hbm.at[nbr_vs[b]], rows_vs[b],
                                    sems[b])

        gather = load_and_gather(0)
        for k in range(n_chunks):
            b = k % 2
            nxt = load_and_gather(k + 1) if k + 1 < n_chunks else None
            gather.wait()
            pltpu.sync_copy(rows_vs[b], acc_sh.at[src_vs[b]], add=True)
            gather = nxt
        plsc.subcore_barrier()
        pltpu.sync_copy(acc_sh.at[pl.ds(r0, rows_per_tile)],
                        out_hbm.at[c, pl.ds(r0, rows_per_tile)])

    return seg_kernel(bonds, src, nbr, init)


def _dense_tc(atoms, partials, kernel_w, bias2d, wn, wi):
    n = atoms.shape[0]
    hid = wn.shape[0]
    blk = n
    grid = n // blk

    def body(atoms_ref, p_ref, kw_ref, bias_ref, wn_ref, wi_ref,
             out_ref):
        a = jnp.dot(atoms_ref[...], kw_ref[...],
                    preferred_element_type=jnp.float32) + bias_ref[...]
        bond = p_ref[0] + p_ref[1]
        rj = lax.broadcasted_iota(jnp.int32, (D, D * D), 0)
        ct = lax.broadcasted_iota(jnp.int32, (D, D * D), 1)
        t_mat = (ct % D == rj).astype(jnp.float32)
        cs = lax.broadcasted_iota(jnp.int32, (D * D, D), 0)
        ri = lax.broadcasted_iota(jnp.int32, (D * D, D), 1)
        s_mat = (cs // D == ri).astype(jnp.float32)
        t = jnp.dot(bond, t_mat, preferred_element_type=jnp.float32)
        agg = jnp.dot(a * t, s_mat, preferred_element_type=jnp.float32)
        nodes = lax.dot_general(agg, wn_ref[...], (((1,), (1,)), ((), ())),
                                preferred_element_type=jnp.float32)
        edges = jnp.maximum(
            lax.dot_general(bond, wi_ref[...], (((1,), (1,)), ((), ())),
                            preferred_element_type=jnp.float32), 0.0)
        out_ref[...] = jnp.maximum(nodes + edges, 0.0)

    return pl.pallas_call(
        body,
        grid=(grid,),
        in_specs=[
            pl.BlockSpec((blk, D), lambda g: (g, 0)),
            pl.BlockSpec((NC, blk, D), lambda g: (0, g, 0)),
            pl.BlockSpec((D, D * D), lambda g: (0, 0)),
            pl.BlockSpec((1, D * D), lambda g: (0, 0)),
            pl.BlockSpec((hid, D), lambda g: (0, 0)),
            pl.BlockSpec((hid, D), lambda g: (0, 0)),
        ],
        out_specs=pl.BlockSpec((blk, hid), lambda g: (g, 0)),
        out_shape=jax.ShapeDtypeStruct((n, hid), jnp.float32),
    )(atoms, partials, kernel_w, bias2d, wn, wi)


def kernel(atoms, bonds, pairs, kernel, bias, weight_node, weight_node_inp):
    n = atoms.shape[0]
    n_pad = ((n + NS * 8 - 1) // (NS * 8)) * (NS * 8)
    zeros = jnp.zeros((NC, n_pad, D), jnp.float32)
    src = pairs[:, 0]
    nbr = pairs[:, 1]
    partials = _segment_sum_sc(bonds, src, nbr, zeros, n)
    result = _dense_tc(atoms, partials, kernel,
                       jnp.reshape(bias, (1, -1)), weight_node,
                       weight_node_inp)
    return (result, result, result, result)


# speedup vs baseline: 1.0073x; 1.0073x over previous
import functools

import jax
import jax.numpy as jnp
from jax import lax
from jax.experimental import pallas as pl
from jax.experimental.pallas import tpu as pltpu
from jax.experimental.pallas import tpu_sc as plsc

NC = 2
NS = 16
D = 16
CHUNK = 2000


def _segment_sum_sc(bonds, src, nbr, init, n_nodes):
    n_edges = src.shape[0]
    n_workers = NC * NS
    epw = n_edges // n_workers
    n_chunks = epw // CHUNK
    n_pad = ((n_nodes + NS * 8 - 1) // (NS * 8)) * (NS * 8)
    rows_per_tile = n_pad // NS

    mesh = plsc.VectorSubcoreMesh(core_axis_name="c", subcore_axis_name="s",
                                  num_cores=NC, num_subcores=NS)

    @functools.partial(
        pl.kernel,
        out_type=jax.ShapeDtypeStruct((NC, n_pad, D), jnp.float32),
        mesh=mesh,
        compiler_params=pltpu.CompilerParams(use_tc_tiling_on_sc=False,
                                             needs_layout_passes=False),
        scratch_types=[
            pltpu.VMEM((CHUNK,), jnp.int32),
            pltpu.VMEM((CHUNK,), jnp.int32),
            pltpu.VMEM((CHUNK,), jnp.int32),
            pltpu.VMEM((CHUNK,), jnp.int32),
            pltpu.VMEM((CHUNK, D), jnp.float32),
            pltpu.VMEM((CHUNK, D), jnp.float32),
            pltpu.VMEM_SHARED((n_pad, D), jnp.float32),
            pltpu.SemaphoreType.DMA,
            pltpu.SemaphoreType.DMA,
        ],
    )
    def seg_kernel(bonds_hbm, src_hbm, nbr_hbm, init_hbm, out_hbm,
                   nbr_v0, nbr_v1, src_v0, src_v1, rows_v0, rows_v1,
                   acc_sh, sem0, sem1):
        nbr_vs = (nbr_v0, nbr_v1)
        src_vs = (src_v0, src_v1)
        rows_vs = (rows_v0, rows_v1)
        sems = (sem0, sem1)
        c = lax.axis_index("c")
        s = lax.axis_index("s")
        r0 = s * rows_per_tile
        pltpu.sync_copy(init_hbm.at[c, pl.ds(r0, rows_per_tile)],
                        acc_sh.at[pl.ds(r0, rows_per_tile)])
        plsc.subcore_barrier()
        base0 = (c * NS + s) * epw

        def load_and_gather(k):
            b = k % 2
            base = base0 + k * CHUNK
            pltpu.sync_copy(nbr_hbm.at[pl.ds(base, CHUNK)], nbr_vs[b])
            pltpu.sync_copy(src_hbm.at[pl.ds(base, CHUNK)], src_vs[b])
            return pltpu.async_copy(bonds_hbm.at[nbr_vs[b]], rows_vs[b],
                                    sems[b])

        gather = load_and_gather(0)
        for k in range(n_chunks):
            b = k % 2
            nxt = load_and_gather(k + 1) if k + 1 < n_chunks else None
            gather.wait()
            pltpu.sync_copy(rows_vs[b], acc_sh.at[src_vs[b]], add=True)
            gather = nxt
        plsc.subcore_barrier()
        pltpu.sync_copy(acc_sh.at[pl.ds(r0, rows_per_tile)],
                        out_hbm.at[c, pl.ds(r0, rows_per_tile)])

    return seg_kernel(bonds, src, nbr, init)


def _dense_tc(atoms, partials, kernel_w, bias2d, wn, wi):
    n = atoms.shape[0]
    hid = wn.shape[0]
    blk = 2000
    grid = n // blk

    def body(atoms_ref, p_ref, kw_ref, bias_ref, wn_ref, wi_ref,
             out_ref):
        a = jnp.dot(atoms_ref[...], kw_ref[...],
                    preferred_element_type=jnp.float32) + bias_ref[...]
        bond = p_ref[0] + p_ref[1]
        rj = lax.broadcasted_iota(jnp.int32, (D, D * D), 0)
        ct = lax.broadcasted_iota(jnp.int32, (D, D * D), 1)
        t_mat = (ct % D == rj).astype(jnp.float32)
        cs = lax.broadcasted_iota(jnp.int32, (D * D, D), 0)
        ri = lax.broadcasted_iota(jnp.int32, (D * D, D), 1)
        s_mat = (cs // D == ri).astype(jnp.float32)
        t = jnp.dot(bond, t_mat, preferred_element_type=jnp.float32)
        agg = jnp.dot(a * t, s_mat, preferred_element_type=jnp.float32)
        nodes = lax.dot_general(agg, wn_ref[...], (((1,), (1,)), ((), ())),
                                preferred_element_type=jnp.float32)
        edges = jnp.maximum(
            lax.dot_general(bond, wi_ref[...], (((1,), (1,)), ((), ())),
                            preferred_element_type=jnp.float32), 0.0)
        out_ref[...] = jnp.maximum(nodes + edges, 0.0)

    return pl.pallas_call(
        body,
        grid=(grid,),
        in_specs=[
            pl.BlockSpec((blk, D), lambda g: (g, 0)),
            pl.BlockSpec((NC, blk, D), lambda g: (0, g, 0)),
            pl.BlockSpec((D, D * D), lambda g: (0, 0)),
            pl.BlockSpec((1, D * D), lambda g: (0, 0)),
            pl.BlockSpec((hid, D), lambda g: (0, 0)),
            pl.BlockSpec((hid, D), lambda g: (0, 0)),
        ],
        out_specs=pl.BlockSpec((blk, hid), lambda g: (g, 0)),
        out_shape=jax.ShapeDtypeStruct((n, hid), jnp.float32),
    )(atoms, partials, kernel_w, bias2d, wn, wi)


def kernel(atoms, bonds, pairs, kernel, bias, weight_node, weight_node_inp):
    n = atoms.shape[0]
    n_pad = ((n + NS * 8 - 1) // (NS * 8)) * (NS * 8)
    zeros = jnp.zeros((NC, n_pad, D), jnp.float32)
    sel0 = jnp.array([1, 0], jnp.int32)
    sel1 = jnp.array([0, 1], jnp.int32)
    src = jnp.sum(pairs * sel0, axis=1)
    nbr = jnp.sum(pairs * sel1, axis=1)
    partials = _segment_sum_sc(bonds, src, nbr, zeros, n)
    result = _dense_tc(atoms, partials, kernel,
                       jnp.reshape(bias, (1, -1)), weight_node,
                       weight_node_inp)
    return (result, result, result, result)


# speedup vs baseline: 1.0095x; 1.0021x over previous
import functools

import jax
import jax.numpy as jnp
from jax import lax
from jax.experimental import pallas as pl
from jax.experimental.pallas import tpu as pltpu
from jax.experimental.pallas import tpu_sc as plsc

NC = 2
NS = 16
D = 16
CHUNK = 2000


def _segment_sum_sc(bonds, src, nbr, n_nodes):
    n_edges = src.shape[0]
    n_workers = NC * NS
    epw = n_edges // n_workers
    n_chunks = epw // CHUNK
    n_pad = ((n_nodes + NS * 8 - 1) // (NS * 8)) * (NS * 8)
    rows_per_tile = n_pad // NS

    mesh = plsc.VectorSubcoreMesh(core_axis_name="c", subcore_axis_name="s",
                                  num_cores=NC, num_subcores=NS)

    @functools.partial(
        pl.kernel,
        out_type=jax.ShapeDtypeStruct((NC, n_pad, D), jnp.float32),
        mesh=mesh,
        compiler_params=pltpu.CompilerParams(use_tc_tiling_on_sc=False,
                                             needs_layout_passes=False),
        scratch_types=[
            pltpu.VMEM((CHUNK,), jnp.int32),
            pltpu.VMEM((CHUNK,), jnp.int32),
            pltpu.VMEM((CHUNK,), jnp.int32),
            pltpu.VMEM((CHUNK,), jnp.int32),
            pltpu.VMEM((CHUNK, D), jnp.float32),
            pltpu.VMEM((CHUNK, D), jnp.float32),
            pltpu.VMEM_SHARED((n_pad, D), jnp.float32),
            pltpu.SemaphoreType.DMA,
            pltpu.SemaphoreType.DMA,
        ],
    )
    def seg_kernel(bonds_hbm, src_hbm, nbr_hbm, out_hbm,
                   nbr_v0, nbr_v1, src_v0, src_v1, rows_v0, rows_v1,
                   acc_sh, sem0, sem1):
        nbr_vs = (nbr_v0, nbr_v1)
        src_vs = (src_v0, src_v1)
        rows_vs = (rows_v0, rows_v1)
        sems = (sem0, sem1)
        c = lax.axis_index("c")
        s = lax.axis_index("s")
        r0 = s * rows_per_tile
        zero16 = jnp.zeros((16,), jnp.float32)

        def zfill(i, carry):
            rows_v0[i, :] = zero16
            return carry

        lax.fori_loop(0, rows_per_tile, zfill, 0)
        pltpu.sync_copy(rows_v0.at[pl.ds(0, rows_per_tile)],
                        acc_sh.at[pl.ds(r0, rows_per_tile)])
        plsc.subcore_barrier()
        base0 = (c * NS + s) * epw

        def load_and_gather(k):
            b = k % 2
            base = base0 + k * CHUNK
            pltpu.sync_copy(nbr_hbm.at[pl.ds(base, CHUNK)], nbr_vs[b])
            pltpu.sync_copy(src_hbm.at[pl.ds(base, CHUNK)], src_vs[b])
            return pltpu.async_copy(bonds_hbm.at[nbr_vs[b]], rows_vs[b],
                                    sems[b])

        gather = load_and_gather(0)
        for k in range(n_chunks):
            b = k % 2
            nxt = load_and_gather(k + 1) if k + 1 < n_chunks else None
            gather.wait()
            pltpu.sync_copy(rows_vs[b], acc_sh.at[src_vs[b]], add=True)
            gather = nxt
        plsc.subcore_barrier()
        pltpu.sync_copy(acc_sh.at[pl.ds(r0, rows_per_tile)],
                        out_hbm.at[c, pl.ds(r0, rows_per_tile)])

    return seg_kernel(bonds, src, nbr)


def _dense_tc(atoms, partials, kernel_w, bias2d, wn, wi):
    n = atoms.shape[0]
    hid = wn.shape[0]
    blk = 2000
    grid = n // blk

    def body(atoms_ref, p_ref, kw_ref, bias_ref, wn_ref, wi_ref,
             out_ref):
        a = jnp.dot(atoms_ref[...], kw_ref[...],
                    preferred_element_type=jnp.float32) + bias_ref[...]
        bond = p_ref[0] + p_ref[1]
        rj = lax.broadcasted_iota(jnp.int32, (D, D * D), 0)
        ct = lax.broadcasted_iota(jnp.int32, (D, D * D), 1)
        t_mat = (ct % D == rj).astype(jnp.float32)
        cs = lax.broadcasted_iota(jnp.int32, (D * D, D), 0)
        ri = lax.broadcasted_iota(jnp.int32, (D * D, D), 1)
        s_mat = (cs // D == ri).astype(jnp.float32)
        t = jnp.dot(bond, t_mat, preferred_element_type=jnp.float32)
        agg = jnp.dot(a * t, s_mat, preferred_element_type=jnp.float32)
        nodes = lax.dot_general(agg, wn_ref[...], (((1,), (1,)), ((), ())),
                                preferred_element_type=jnp.float32)
        edges = jnp.maximum(
            lax.dot_general(bond, wi_ref[...], (((1,), (1,)), ((), ())),
                            preferred_element_type=jnp.float32), 0.0)
        out_ref[...] = jnp.maximum(nodes + edges, 0.0)

    return pl.pallas_call(
        body,
        grid=(grid,),
        in_specs=[
            pl.BlockSpec((blk, D), lambda g: (g, 0)),
            pl.BlockSpec((NC, blk, D), lambda g: (0, g, 0)),
            pl.BlockSpec((D, D * D), lambda g: (0, 0)),
            pl.BlockSpec((1, D * D), lambda g: (0, 0)),
            pl.BlockSpec((hid, D), lambda g: (0, 0)),
            pl.BlockSpec((hid, D), lambda g: (0, 0)),
        ],
        out_specs=pl.BlockSpec((blk, hid), lambda g: (g, 0)),
        out_shape=jax.ShapeDtypeStruct((n, hid), jnp.float32),
    )(atoms, partials, kernel_w, bias2d, wn, wi)


def kernel(atoms, bonds, pairs, kernel, bias, weight_node, weight_node_inp):
    n = atoms.shape[0]
    src = pairs[:, 0]
    nbr = pairs[:, 1]
    partials = _segment_sum_sc(bonds, src, nbr, n)
    result = _dense_tc(atoms, partials, kernel,
                       jnp.reshape(bias, (1, -1)), weight_node,
                       weight_node_inp)
    return (result, result, result, result)
